# Initial kernel scaffold; baseline (speedup 1.0000x reference)
import jax
import jax.numpy as jnp
from jax.experimental import pallas as pl


def kernel(x, embeddings):
    raise NotImplementedError("write your pallas kernel here")


# speedup vs baseline: 7.7579x; 7.7579x over previous
import functools

import jax
import jax.numpy as jnp
from jax import lax
from jax.experimental import pallas as pl
from jax.experimental.pallas import tpu as pltpu
from jax.experimental.pallas import tpu_sc as plsc

_N_TOK = 4096
_N_CLU = 1024
_D = 64
_TB = 512


def _argmin_body(x_ref, et_ref, k_ref):
    x = x_ref[...]
    et = et_ref[...]
    esq = jnp.sum(et * et, axis=0, keepdims=True)
    dots = lax.dot_general(x, et, (((1,), (0,)), ((), ())),
                           precision=lax.Precision.HIGHEST,
                           preferred_element_type=jnp.float32)
    scores = esq - 2.0 * dots
    k = jnp.argmin(scores, axis=1)
    k_ref[0, 0, :] = k.astype(jnp.int32)


def _argmin_call(x, embeddings):
    grid = _N_TOK // _TB
    et = embeddings.T
    return pl.pallas_call(
        _argmin_body,
        grid=(grid,),
        in_specs=[
            pl.BlockSpec((_TB, _D), lambda i: (i, 0)),
            pl.BlockSpec((_D, _N_CLU), lambda i: (0, 0)),
        ],
        out_specs=pl.BlockSpec((1, 1, _TB), lambda i: (i, 0, 0)),
        out_shape=jax.ShapeDtypeStruct((grid, 1, _TB), jnp.int32),
    )(x, et)


_DP = 128


@functools.partial(jax.jit, static_argnames=())
def _gather_rows(table_padded, idx):
    info = plsc.get_sparse_core_info()
    nw = info.num_cores * info.num_subcores
    b_per_w = _N_TOK // nw
    mesh = plsc.VectorSubcoreMesh(core_axis_name="c", subcore_axis_name="s")

    @functools.partial(
        pl.kernel, mesh=mesh,
        out_type=jax.ShapeDtypeStruct((_N_TOK, _DP), jnp.float32),
        scratch_types=[
            pltpu.VMEM((b_per_w,), jnp.int32),
            pltpu.VMEM((b_per_w, _DP), jnp.float32),
            pltpu.SemaphoreType.DMA,
        ],
    )
    def gather(table_hbm, idx_hbm, out_hbm, idx_v, rows_v, sem):
        wid = lax.axis_index("s") * info.num_cores + lax.axis_index("c")
        base = wid * b_per_w
        pltpu.sync_copy(idx_hbm.at[pl.ds(base, b_per_w)], idx_v)
        pltpu.async_copy(table_hbm.at[idx_v], rows_v, sem).wait()
        pltpu.sync_copy(rows_v, out_hbm.at[pl.ds(base, b_per_w)])

    return gather(table_padded, idx)


def kernel(x, embeddings):
    k = _argmin_call(x, embeddings).reshape(_N_TOK)
    table_padded = jnp.pad(embeddings, ((0, 0), (0, _DP - _D)))
    z_q = _gather_rows(table_padded, k)[:, :_D]
    return (z_q, x, z_q)


# speedup vs baseline: 7.7832x; 1.0033x over previous
import functools

import jax
import jax.numpy as jnp
from jax import lax
from jax.experimental import pallas as pl
from jax.experimental.pallas import tpu as pltpu
from jax.experimental.pallas import tpu_sc as plsc

_N_TOK = 4096
_N_CLU = 1024
_D = 64
_DP = 128
_TB = 512
_IDX_ROWS = 32


def _argmin_body(x_ref, et_ref, e_ref, k_ref, tab_ref):
    x = x_ref[...]
    et = et_ref[...]
    esq = jnp.sum(et * et, axis=0, keepdims=True)
    dots = lax.dot_general(x, et, (((1,), (0,)), ((), ())),
                           precision=lax.Precision.HIGHEST,
                           preferred_element_type=jnp.float32)
    scores = esq - 2.0 * dots
    k = jnp.argmin(scores, axis=1).astype(jnp.int32)
    k_ref[...] = k.reshape(1, _TB // 128, 128)

    @pl.when(pl.program_id(0) == 0)
    def _build_padded_table():
        tab_ref[:, : _D] = e_ref[...]
        tab_ref[:, _D:] = jnp.zeros((_N_CLU, _DP - _D), jnp.float32)


def _argmin_call(x, embeddings):
    grid = _N_TOK // _TB
    et = embeddings.T
    return pl.pallas_call(
        _argmin_body,
        grid=(grid,),
        in_specs=[
            pl.BlockSpec((_TB, _D), lambda i: (i, 0)),
            pl.BlockSpec((_D, _N_CLU), lambda i: (0, 0)),
            pl.BlockSpec((_N_CLU, _D), lambda i: (0, 0)),
        ],
        out_specs=[
            pl.BlockSpec((1, _TB // 128, 128), lambda i: (i, 0, 0)),
            pl.BlockSpec((_N_CLU, _DP), lambda i: (0, 0)),
        ],
        out_shape=[
            jax.ShapeDtypeStruct((_N_TOK // _TB, _TB // 128, 128), jnp.int32),
            jax.ShapeDtypeStruct((_N_CLU, _DP), jnp.float32),
        ],
    )(x, et, embeddings)


@functools.partial(jax.jit, static_argnames=())
def _gather_rows(table_padded, idx):
    info = plsc.get_sparse_core_info()
    nw = info.num_cores * info.num_subcores
    b_per_w = _N_TOK // nw
    mesh = plsc.VectorSubcoreMesh(core_axis_name="c", subcore_axis_name="s")

    @functools.partial(
        pl.kernel, mesh=mesh,
        out_type=jax.ShapeDtypeStruct((_N_TOK, _DP), jnp.float32),
        scratch_types=[
            pltpu.VMEM((b_per_w,), jnp.int32),
            pltpu.VMEM((b_per_w, _DP), jnp.float32),
            pltpu.SemaphoreType.DMA,
        ],
    )
    def gather(table_hbm, idx_hbm, out_hbm, idx_v, rows_v, sem):
        wid = lax.axis_index("s") * info.num_cores + lax.axis_index("c")
        base = wid * b_per_w
        pltpu.sync_copy(idx_hbm.at[wid // 4, wid % 4], idx_v)
        pltpu.async_copy(table_hbm.at[idx_v], rows_v, sem).wait()
        pltpu.sync_copy(rows_v, out_hbm.at[pl.ds(base, b_per_w)])

    return gather(table_padded, idx)


def kernel(x, embeddings):
    k, table_padded = _argmin_call(x, embeddings)
    z_q = _gather_rows(table_padded, k)[:, :_D]
    return (z_q, x, z_q)
